# baseline (speedup 1.0000x reference)
import functools

import jax
import jax.numpy as jnp
from jax import lax
from jax.experimental import pallas as pl
from jax.experimental.pallas import tpu as pltpu
from jax.experimental.pallas import tpu_sc as plsc

N_NODES = 10000
D_IN = 128
H = 16
E_EDGES = 320000

NC = 2
NS = 16
CHUNK = 256
NCH = 79
EP_TILE = NCH * CHUNK
EP = EP_TILE * NS
NROWS = 10240
ZROWS = NROWS // NS
TRASH = 10000


def _matmul16(a, w):
    return jnp.dot(a, w, preferred_element_type=jnp.float32,
                   precision=lax.Precision.HIGHEST)


def _proj1_body(xp_ref, xu_ref, wvl_ref, wrl_ref, wvr_ref, wrr_ref,
                y_ref, r_ref):
    xp = xp_ref[...]
    xu = xu_ref[...]
    y_ref[0] = _matmul16(xp, wvl_ref[...])
    y_ref[1] = _matmul16(xu, wrl_ref[...])
    r_ref[0] = _matmul16(xu, wvr_ref[...])
    r_ref[1] = _matmul16(xp, wrr_ref[...])


def _proj1(x_p, x_u, wvl, wrl, wvr, wrr):
    blk = 2000
    grid = N_NODES // blk
    wspec = pl.BlockSpec((D_IN, H), lambda i: (0, 0))
    return pl.pallas_call(
        _proj1_body,
        grid=(grid,),
        in_specs=[
            pl.BlockSpec((blk, D_IN), lambda i: (i, 0)),
            pl.BlockSpec((blk, D_IN), lambda i: (i, 0)),
            wspec, wspec, wspec, wspec,
        ],
        out_specs=[
            pl.BlockSpec((2, blk, H), lambda i: (0, i, 0)),
            pl.BlockSpec((2, blk, H), lambda i: (0, i, 0)),
        ],
        out_shape=[
            jax.ShapeDtypeStruct((2, NROWS, H), jnp.float32),
            jax.ShapeDtypeStruct((2, NROWS, H), jnp.float32),
        ],
    )(x_p, x_u, wvl, wrl, wvr, wrr)


def _pipeline_segsum(y_hbm, src_v, dst_v, rows0, rows1, acc_sh,
                     semg0, semg1, sems0, sems1, count=None):
    if count is not None:
        ones_v, cnt_sh, semc = count

    def fire_gather(j, rows, semg):
        pltpu.async_copy(y_hbm.at[src_v.at[j]], rows, semg)

    def wait_gather(j, rows, semg):
        pltpu.make_async_copy(y_hbm.at[src_v.at[j]], rows, semg).wait()

    def fire_scatter(j, rows, sems):
        pltpu.async_copy(rows, acc_sh.at[dst_v.at[j]], sems, add=True)

    def wait_scatter(j, rows, sems):
        pltpu.make_async_copy(rows, acc_sh.at[dst_v.at[j]], sems).wait()

    def fire_count(j):
        pltpu.async_copy(ones_v, cnt_sh.at[dst_v.at[j]], semc, add=True)

    def wait_count(j):
        pltpu.make_async_copy(ones_v, cnt_sh.at[dst_v.at[j]], semc).wait()

    fire_gather(0, rows0, semg0)
    wait_gather(0, rows0, semg0)
    fire_scatter(0, rows0, sems0)
    if count is not None:
        fire_count(0)
    fire_gather(1, rows1, semg1)

    def step(j, a, b):
        (rows_a, semg_a, sems_a) = a
        (rows_b, semg_b, sems_b) = b
        wait_gather(j, rows_a, semg_a)
        fire_scatter(j, rows_a, sems_a)
        if count is not None:
            fire_count(j)
            wait_count(j - 1)
        wait_scatter(j - 1, rows_b, sems_b)

        @pl.when(j + 1 < NCH)
        def _():
            fire_gather(j + 1, rows_b, semg_b)

    buf0 = (rows0, semg0, sems0)
    buf1 = (rows1, semg1, sems1)

    def pair_body(p, carry):
        step(2 * p + 1, buf1, buf0)
        step(2 * p + 2, buf0, buf1)
        return carry

    lax.fori_loop(0, (NCH - 1) // 2, pair_body, 0)
    wait_scatter(NCH - 1, rows0, sems0)
    if count is not None:
        wait_count(NCH - 1)


def _node_state(acc_v, cnt_v, root_v, bias, g, k):
    cvec = jnp.maximum(cnt_v[pl.ds(g * 16, 16)], 1.0)
    r = g * 16 + k
    return jnp.maximum(acc_v[r] / cvec[k] + root_v[r] + bias, 0.0), r


@functools.lru_cache(maxsize=None)
def _make_layer1():
    out_type = [
        jax.ShapeDtypeStruct((NC, NROWS, H), jnp.float32),
        jax.ShapeDtypeStruct((NC, NROWS, H), jnp.float32),
        jax.ShapeDtypeStruct((NC, NROWS), jnp.float32),
    ]
    scratch = [
        pltpu.VMEM((NCH, CHUNK), jnp.int32),
        pltpu.VMEM((NCH, CHUNK), jnp.int32),
        pltpu.VMEM((CHUNK, H), jnp.float32),
        pltpu.VMEM((CHUNK, H), jnp.float32),
        pltpu.VMEM_SHARED((NROWS, H), jnp.float32),
        pltpu.SemaphoreType.DMA,
        pltpu.SemaphoreType.DMA,
        pltpu.SemaphoreType.DMA,
        pltpu.SemaphoreType.DMA,
        pltpu.SemaphoreType.DMA,
        pltpu.VMEM((CHUNK,), jnp.float32),
        pltpu.VMEM_SHARED((NROWS,), jnp.float32),
        pltpu.VMEM((ZROWS, H), jnp.float32),
        pltpu.VMEM((ZROWS, H), jnp.float32),
        pltpu.VMEM((ZROWS,), jnp.float32),
        pltpu.VMEM((H,), jnp.float32),
        pltpu.VMEM((H, H), jnp.float32),
        pltpu.VMEM((H, H), jnp.float32),
        pltpu.VMEM((ZROWS, H), jnp.float32),
        pltpu.VMEM((ZROWS, H), jnp.float32),
    ]

    def body(y_hbm, src_hbm, dst_hbm, zrow_hbm, z1_hbm, root_hbm, b_hbm,
             wy_hbm, wr_hbm, y2_out, r2_out, cnt_out,
             src_v, dst_v, rows0, rows1, acc_sh, semg0, semg1, sems0,
             sems1, semc, ones_v, cnt_sh, acc_v, root_v, cnt_v, b_v,
             wy_v, wr_v, y2_v, r2_v):
        c = lax.axis_index("c")
        s = lax.axis_index("s")
        base = s * ZROWS

        pltpu.sync_copy(src_hbm.at[c].at[s], src_v)
        pltpu.sync_copy(dst_hbm.at[c].at[s], dst_v)
        pltpu.sync_copy(zrow_hbm, acc_sh.at[pl.ds(base, ZROWS)])
        pltpu.sync_copy(z1_hbm, cnt_sh.at[pl.ds(base, ZROWS)])
        pltpu.sync_copy(root_hbm.at[c].at[pl.ds(base, ZROWS)], root_v)
        pltpu.sync_copy(b_hbm.at[c], b_v)
        pltpu.sync_copy(wy_hbm.at[c], wy_v)
        pltpu.sync_copy(wr_hbm.at[c], wr_v)
        for i in range(CHUNK // 16):
            ones_v[pl.ds(i * 16, 16)] = jnp.ones((16,), jnp.float32)
        plsc.subcore_barrier()

        _pipeline_segsum(y_hbm, src_v, dst_v, rows0, rows1, acc_sh,
                         semg0, semg1, sems0, sems1,
                         count=(ones_v, cnt_sh, semc))
        plsc.subcore_barrier()

        pltpu.sync_copy(cnt_sh.at[pl.ds(base, ZROWS)],
                        cnt_out.at[c].at[pl.ds(base, ZROWS)])
        pltpu.sync_copy(cnt_sh.at[pl.ds(base, ZROWS)], cnt_v)
        pltpu.sync_copy(acc_sh.at[pl.ds(base, ZROWS)], acc_v)
        bias = b_v[...]

        def grp_body(g, carry):
            for k in range(16):
                u, r = _node_state(acc_v, cnt_v, root_v, bias, g, k)
                y2a = u[0] * wy_v[0]
                r2a = u[0] * wr_v[0]
                for k2 in range(1, 16):
                    y2a = y2a + u[k2] * wy_v[k2]
                    r2a = r2a + u[k2] * wr_v[k2]
                y2_v[r] = y2a
                r2_v[r] = r2a
            return carry

        lax.fori_loop(0, ZROWS // 16, grp_body, 0)
        pltpu.sync_copy(y2_v, y2_out.at[1 - c].at[pl.ds(base, ZROWS)])
        pltpu.sync_copy(r2_v, r2_out.at[c].at[pl.ds(base, ZROWS)])

    mesh = plsc.VectorSubcoreMesh(core_axis_name="c", subcore_axis_name="s",
                                  num_cores=NC, num_subcores=NS)
    return pl.kernel(body, out_type=out_type, mesh=mesh,
                     scratch_types=scratch,
                     compiler_params=pltpu.CompilerParams(
                         use_tc_tiling_on_sc=False))


@functools.lru_cache(maxsize=None)
def _make_layer2():
    out_type = [jax.ShapeDtypeStruct((NC, NROWS, H), jnp.float32)]
    scratch = [
        pltpu.VMEM((NCH, CHUNK), jnp.int32),
        pltpu.VMEM((NCH, CHUNK), jnp.int32),
        pltpu.VMEM((CHUNK, H), jnp.float32),
        pltpu.VMEM((CHUNK, H), jnp.float32),
        pltpu.VMEM_SHARED((NROWS, H), jnp.float32),
        pltpu.SemaphoreType.DMA,
        pltpu.SemaphoreType.DMA,
        pltpu.SemaphoreType.DMA,
        pltpu.SemaphoreType.DMA,
        pltpu.VMEM((ZROWS, H), jnp.float32),
        pltpu.VMEM((ZROWS, H), jnp.float32),
        pltpu.VMEM((ZROWS,), jnp.float32),
        pltpu.VMEM((H,), jnp.float32),
    ]

    def body(y_hbm, src_hbm, dst_hbm, zrow_hbm, cnt_hbm, root_hbm, b_hbm,
             s_out, src_v, dst_v, rows0, rows1, acc_sh, semg0, semg1,
             sems0, sems1, acc_v, root_v, cnt_v, b_v):
        c = lax.axis_index("c")
        s = lax.axis_index("s")
        base = s * ZROWS

        pltpu.sync_copy(src_hbm.at[c].at[s], src_v)
        pltpu.sync_copy(dst_hbm.at[c].at[s], dst_v)
        pltpu.sync_copy(zrow_hbm, acc_sh.at[pl.ds(base, ZROWS)])
        pltpu.sync_copy(cnt_hbm.at[c].at[pl.ds(base, ZROWS)], cnt_v)
        pltpu.sync_copy(root_hbm.at[c].at[pl.ds(base, ZROWS)], root_v)
        pltpu.sync_copy(b_hbm.at[c], b_v)
        plsc.subcore_barrier()

        _pipeline_segsum(y_hbm, src_v, dst_v, rows0, rows1, acc_sh,
                         semg0, semg1, sems0, sems1)
        plsc.subcore_barrier()

        pltpu.sync_copy(acc_sh.at[pl.ds(base, ZROWS)], acc_v)
        bias = b_v[...]

        def grp_body(g, carry):
            for k in range(16):
                u, r = _node_state(acc_v, cnt_v, root_v, bias, g, k)
                acc_v[r] = u
            return carry

        lax.fori_loop(0, ZROWS // 16, grp_body, 0)
        pltpu.sync_copy(acc_v, s_out.at[c].at[pl.ds(base, ZROWS)])

    mesh = plsc.VectorSubcoreMesh(core_axis_name="c", subcore_axis_name="s",
                                  num_cores=NC, num_subcores=NS)
    return pl.kernel(body, out_type=out_type, mesh=mesh,
                     scratch_types=scratch,
                     compiler_params=pltpu.CompilerParams(
                         use_tc_tiling_on_sc=False))


def _pad_edges(idx, fill):
    pad = jnp.full((EP - E_EDGES,), fill, jnp.int32)
    return jnp.concatenate([idx.astype(jnp.int32), pad])


def kernel(x_Persona, x_Ubicacion, edge_index_visits, edge_index_rev,
           W1v_l, b1v, W1v_r, W1r_l, b1r, W1r_r,
           W2v_l, b2v, W2v_r, W2r_l, b2r, W2r_r):
    src_all = jnp.stack([
        _pad_edges(edge_index_visits[0], 0),
        _pad_edges(edge_index_rev[0] + NROWS, NROWS),
    ]).reshape(NC, NS, NCH, CHUNK)
    dst_all = jnp.stack([
        _pad_edges(edge_index_visits[1], TRASH),
        _pad_edges(edge_index_rev[1], TRASH),
    ]).reshape(NC, NS, NCH, CHUNK)
    zrow = jnp.zeros((ZROWS, H), jnp.float32)
    z1 = jnp.zeros((ZROWS,), jnp.float32)
    b1 = jnp.stack([b1v, b1r])
    b2 = jnp.stack([b2v, b2r])
    wy = jnp.stack([W2r_l, W2v_l])
    wr = jnp.stack([W2v_r, W2r_r])

    y1, r1 = _proj1(x_Persona, x_Ubicacion, W1v_l, W1r_l, W1v_r, W1r_r)

    y2, r2, cnt = _make_layer1()(y1.reshape(NC * NROWS, H), src_all,
                                 dst_all, zrow, z1, r1, b1, wy, wr)

    (out,) = _make_layer2()(y2.reshape(NC * NROWS, H), src_all, dst_all,
                            zrow, cnt, r2, b2)
    return (out[1, :N_NODES], out[0, :N_NODES])


# speedup vs baseline: 1.1726x; 1.1726x over previous
import functools

import jax
import jax.numpy as jnp
from jax import lax
from jax.experimental import pallas as pl
from jax.experimental.pallas import tpu as pltpu
from jax.experimental.pallas import tpu_sc as plsc

N_NODES = 10000
D_IN = 128
H = 16
E_EDGES = 320000

NC = 2
NS = 16
CHUNK = 256
NCH = 79
EP_TILE = NCH * CHUNK
EP = EP_TILE * NS
NROWS = 10240
ZROWS = NROWS // NS
TRASH = 10000


def _matmul16(a, w):
    return jnp.dot(a, w, preferred_element_type=jnp.float32,
                   precision=lax.Precision.HIGHEST)


def _proj1_body(xp_ref, xu_ref, wvl_ref, wrl_ref, wvr_ref, wrr_ref,
                y_ref, r_ref):
    xp = xp_ref[...]
    xu = xu_ref[...]
    y_ref[0] = _matmul16(xp, wvl_ref[...])
    y_ref[1] = _matmul16(xu, wrl_ref[...])
    r_ref[0] = _matmul16(xu, wvr_ref[...])
    r_ref[1] = _matmul16(xp, wrr_ref[...])


def _proj1(x_p, x_u, wvl, wrl, wvr, wrr):
    blk = 2000
    grid = N_NODES // blk
    wspec = pl.BlockSpec((D_IN, H), lambda i: (0, 0))
    return pl.pallas_call(
        _proj1_body,
        grid=(grid,),
        in_specs=[
            pl.BlockSpec((blk, D_IN), lambda i: (i, 0)),
            pl.BlockSpec((blk, D_IN), lambda i: (i, 0)),
            wspec, wspec, wspec, wspec,
        ],
        out_specs=[
            pl.BlockSpec((2, blk, H), lambda i: (0, i, 0)),
            pl.BlockSpec((2, blk, H), lambda i: (0, i, 0)),
        ],
        out_shape=[
            jax.ShapeDtypeStruct((2, NROWS, H), jnp.float32),
            jax.ShapeDtypeStruct((2, NROWS, H), jnp.float32),
        ],
    )(x_p, x_u, wvl, wrl, wvr, wrr)


def _pipeline_segsum(y_hbm, src_v, dst_v, rows0, rows1, acc_sh,
                     semg0, semg1, sems0, sems1, count=None):
    if count is not None:
        ones_v, cnt_sh, semc = count

    def fire_gather(j, rows, semg):
        pltpu.async_copy(y_hbm.at[src_v.at[j]], rows, semg)

    def wait_gather(j, rows, semg):
        pltpu.make_async_copy(y_hbm.at[src_v.at[j]], rows, semg).wait()

    def fire_scatter(j, rows, sems):
        pltpu.async_copy(rows, acc_sh.at[dst_v.at[j]], sems, add=True)

    def wait_scatter(j, rows, sems):
        pltpu.make_async_copy(rows, acc_sh.at[dst_v.at[j]], sems).wait()

    def fire_count(j):
        pltpu.async_copy(ones_v, cnt_sh.at[dst_v.at[j]], semc, add=True)

    def wait_count(j):
        pltpu.make_async_copy(ones_v, cnt_sh.at[dst_v.at[j]], semc).wait()

    def drain_scatter(j, rows, semg):
        wait_gather(j, rows, semg)
        pltpu.sync_copy(rows, acc_sh.at[dst_v.at[j]], add=True)
        if count is not None:
            fire_count(j)

            @pl.when(j >= 1)
            def _():
                wait_count(j - 1)

    fire_gather(0, rows0, semg0)

    def pair_body(p, carry):
        j0 = 2 * p
        fire_gather(j0 + 1, rows1, semg1)
        drain_scatter(j0, rows0, semg0)
        fire_gather(j0 + 2, rows0, semg0)
        drain_scatter(j0 + 1, rows1, semg1)
        return carry

    lax.fori_loop(0, (NCH - 1) // 2, pair_body, 0)
    drain_scatter(NCH - 1, rows0, semg0)
    if count is not None:
        wait_count(NCH - 1)


def _node_state(acc_v, cnt_v, root_v, bias, g, k):
    cvec = jnp.maximum(cnt_v[pl.ds(g * 16, 16)], 1.0)
    r = g * 16 + k
    return jnp.maximum(acc_v[r] / cvec[k] + root_v[r] + bias, 0.0), r


@functools.lru_cache(maxsize=None)
def _make_layer1():
    out_type = [
        jax.ShapeDtypeStruct((NC, NROWS, H), jnp.float32),
        jax.ShapeDtypeStruct((NC, NROWS, H), jnp.float32),
        jax.ShapeDtypeStruct((NC, NROWS), jnp.float32),
    ]
    scratch = [
        pltpu.VMEM((NCH, CHUNK), jnp.int32),
        pltpu.VMEM((NCH, CHUNK), jnp.int32),
        pltpu.VMEM((CHUNK, H), jnp.float32),
        pltpu.VMEM((CHUNK, H), jnp.float32),
        pltpu.VMEM_SHARED((NROWS, H), jnp.float32),
        pltpu.SemaphoreType.DMA,
        pltpu.SemaphoreType.DMA,
        pltpu.SemaphoreType.DMA,
        pltpu.SemaphoreType.DMA,
        pltpu.SemaphoreType.DMA,
        pltpu.VMEM((CHUNK,), jnp.float32),
        pltpu.VMEM_SHARED((NROWS,), jnp.float32),
        pltpu.VMEM((ZROWS, H), jnp.float32),
        pltpu.VMEM((ZROWS, H), jnp.float32),
        pltpu.VMEM((ZROWS,), jnp.float32),
        pltpu.VMEM((H,), jnp.float32),
        pltpu.VMEM((H, H), jnp.float32),
        pltpu.VMEM((H, H), jnp.float32),
        pltpu.VMEM((ZROWS, H), jnp.float32),
        pltpu.VMEM((ZROWS, H), jnp.float32),
    ]

    def body(y_hbm, src_hbm, dst_hbm, zrow_hbm, z1_hbm, root_hbm, b_hbm,
             wy_hbm, wr_hbm, y2_out, r2_out, cnt_out,
             src_v, dst_v, rows0, rows1, acc_sh, semg0, semg1, sems0,
             sems1, semc, ones_v, cnt_sh, acc_v, root_v, cnt_v, b_v,
             wy_v, wr_v, y2_v, r2_v):
        c = lax.axis_index("c")
        s = lax.axis_index("s")
        base = s * ZROWS

        pltpu.sync_copy(src_hbm.at[c].at[s], src_v)
        pltpu.sync_copy(dst_hbm.at[c].at[s], dst_v)
        pltpu.sync_copy(zrow_hbm, acc_sh.at[pl.ds(base, ZROWS)])
        pltpu.sync_copy(z1_hbm, cnt_sh.at[pl.ds(base, ZROWS)])
        pltpu.sync_copy(root_hbm.at[c].at[pl.ds(base, ZROWS)], root_v)
        pltpu.sync_copy(b_hbm.at[c], b_v)
        pltpu.sync_copy(wy_hbm.at[c], wy_v)
        pltpu.sync_copy(wr_hbm.at[c], wr_v)
        for i in range(CHUNK // 16):
            ones_v[pl.ds(i * 16, 16)] = jnp.ones((16,), jnp.float32)
        plsc.subcore_barrier()

        _pipeline_segsum(y_hbm, src_v, dst_v, rows0, rows1, acc_sh,
                         semg0, semg1, sems0, sems1,
                         count=(ones_v, cnt_sh, semc))
        plsc.subcore_barrier()

        pltpu.sync_copy(cnt_sh.at[pl.ds(base, ZROWS)],
                        cnt_out.at[c].at[pl.ds(base, ZROWS)])
        pltpu.sync_copy(cnt_sh.at[pl.ds(base, ZROWS)], cnt_v)
        pltpu.sync_copy(acc_sh.at[pl.ds(base, ZROWS)], acc_v)
        bias = b_v[...]

        def grp_body(g, carry):
            for k in range(16):
                u, r = _node_state(acc_v, cnt_v, root_v, bias, g, k)
                y2a = u[0] * wy_v[0]
                r2a = u[0] * wr_v[0]
                for k2 in range(1, 16):
                    y2a = y2a + u[k2] * wy_v[k2]
                    r2a = r2a + u[k2] * wr_v[k2]
                y2_v[r] = y2a
                r2_v[r] = r2a
            return carry

        lax.fori_loop(0, ZROWS // 16, grp_body, 0)
        pltpu.sync_copy(y2_v, y2_out.at[1 - c].at[pl.ds(base, ZROWS)])
        pltpu.sync_copy(r2_v, r2_out.at[c].at[pl.ds(base, ZROWS)])

    mesh = plsc.VectorSubcoreMesh(core_axis_name="c", subcore_axis_name="s",
                                  num_cores=NC, num_subcores=NS)
    return pl.kernel(body, out_type=out_type, mesh=mesh,
                     scratch_types=scratch,
                     compiler_params=pltpu.CompilerParams(
                         use_tc_tiling_on_sc=False))


@functools.lru_cache(maxsize=None)
def _make_layer2():
    out_type = [jax.ShapeDtypeStruct((NC, NROWS, H), jnp.float32)]
    scratch = [
        pltpu.VMEM((NCH, CHUNK), jnp.int32),
        pltpu.VMEM((NCH, CHUNK), jnp.int32),
        pltpu.VMEM((CHUNK, H), jnp.float32),
        pltpu.VMEM((CHUNK, H), jnp.float32),
        pltpu.VMEM_SHARED((NROWS, H), jnp.float32),
        pltpu.SemaphoreType.DMA,
        pltpu.SemaphoreType.DMA,
        pltpu.SemaphoreType.DMA,
        pltpu.SemaphoreType.DMA,
        pltpu.VMEM((ZROWS, H), jnp.float32),
        pltpu.VMEM((ZROWS, H), jnp.float32),
        pltpu.VMEM((ZROWS,), jnp.float32),
        pltpu.VMEM((H,), jnp.float32),
    ]

    def body(y_hbm, src_hbm, dst_hbm, zrow_hbm, cnt_hbm, root_hbm, b_hbm,
             s_out, src_v, dst_v, rows0, rows1, acc_sh, semg0, semg1,
             sems0, sems1, acc_v, root_v, cnt_v, b_v):
        c = lax.axis_index("c")
        s = lax.axis_index("s")
        base = s * ZROWS

        pltpu.sync_copy(src_hbm.at[c].at[s], src_v)
        pltpu.sync_copy(dst_hbm.at[c].at[s], dst_v)
        pltpu.sync_copy(zrow_hbm, acc_sh.at[pl.ds(base, ZROWS)])
        pltpu.sync_copy(cnt_hbm.at[c].at[pl.ds(base, ZROWS)], cnt_v)
        pltpu.sync_copy(root_hbm.at[c].at[pl.ds(base, ZROWS)], root_v)
        pltpu.sync_copy(b_hbm.at[c], b_v)
        plsc.subcore_barrier()

        _pipeline_segsum(y_hbm, src_v, dst_v, rows0, rows1, acc_sh,
                         semg0, semg1, sems0, sems1)
        plsc.subcore_barrier()

        pltpu.sync_copy(acc_sh.at[pl.ds(base, ZROWS)], acc_v)
        bias = b_v[...]

        def grp_body(g, carry):
            for k in range(16):
                u, r = _node_state(acc_v, cnt_v, root_v, bias, g, k)
                acc_v[r] = u
            return carry

        lax.fori_loop(0, ZROWS // 16, grp_body, 0)
        pltpu.sync_copy(acc_v, s_out.at[c].at[pl.ds(base, ZROWS)])

    mesh = plsc.VectorSubcoreMesh(core_axis_name="c", subcore_axis_name="s",
                                  num_cores=NC, num_subcores=NS)
    return pl.kernel(body, out_type=out_type, mesh=mesh,
                     scratch_types=scratch,
                     compiler_params=pltpu.CompilerParams(
                         use_tc_tiling_on_sc=False))


def _pad_edges(idx, fill):
    pad = jnp.full((EP - E_EDGES,), fill, jnp.int32)
    return jnp.concatenate([idx.astype(jnp.int32), pad])


def kernel(x_Persona, x_Ubicacion, edge_index_visits, edge_index_rev,
           W1v_l, b1v, W1v_r, W1r_l, b1r, W1r_r,
           W2v_l, b2v, W2v_r, W2r_l, b2r, W2r_r):
    src_all = jnp.stack([
        _pad_edges(edge_index_visits[0], 0),
        _pad_edges(edge_index_rev[0] + NROWS, NROWS),
    ]).reshape(NC, NS, NCH, CHUNK)
    dst_all = jnp.stack([
        _pad_edges(edge_index_visits[1], TRASH),
        _pad_edges(edge_index_rev[1], TRASH),
    ]).reshape(NC, NS, NCH, CHUNK)
    zrow = jnp.zeros((ZROWS, H), jnp.float32)
    z1 = jnp.zeros((ZROWS,), jnp.float32)
    b1 = jnp.stack([b1v, b1r])
    b2 = jnp.stack([b2v, b2r])
    wy = jnp.stack([W2r_l, W2v_l])
    wr = jnp.stack([W2v_r, W2r_r])

    y1, r1 = _proj1(x_Persona, x_Ubicacion, W1v_l, W1r_l, W1v_r, W1r_r)

    y2, r2, cnt = _make_layer1()(y1.reshape(NC * NROWS, H), src_all,
                                 dst_all, zrow, z1, r1, b1, wy, wr)

    (out,) = _make_layer2()(y2.reshape(NC * NROWS, H), src_all, dst_all,
                            zrow, cnt, r2, b2)
    return (out[1, :N_NODES], out[0, :N_NODES])


# speedup vs baseline: 1.1788x; 1.0053x over previous
import functools

import jax
import jax.numpy as jnp
from jax import lax
from jax.experimental import pallas as pl
from jax.experimental.pallas import tpu as pltpu
from jax.experimental.pallas import tpu_sc as plsc

N_NODES = 10000
D_IN = 128
H = 16
E_EDGES = 320000

NC = 2
NS = 16
CHUNK = 256
NCH = 79
EP_TILE = NCH * CHUNK
EP = EP_TILE * NS
NROWS = 10240
ZROWS = NROWS // NS
TRASH = 10000


def _matmul16(a, w):
    return jnp.dot(a, w, preferred_element_type=jnp.float32,
                   precision=lax.Precision.HIGHEST)


def _proj1_body(xp_ref, xu_ref, wvl_ref, wrl_ref, wvr_ref, wrr_ref,
                y_ref, r_ref):
    xp = xp_ref[...]
    xu = xu_ref[...]
    y_ref[0] = _matmul16(xp, wvl_ref[...])
    y_ref[1] = _matmul16(xu, wrl_ref[...])
    r_ref[0] = _matmul16(xu, wvr_ref[...])
    r_ref[1] = _matmul16(xp, wrr_ref[...])


def _proj1(x_p, x_u, wvl, wrl, wvr, wrr):
    blk = 2000
    grid = N_NODES // blk
    wspec = pl.BlockSpec((D_IN, H), lambda i: (0, 0))
    return pl.pallas_call(
        _proj1_body,
        grid=(grid,),
        in_specs=[
            pl.BlockSpec((blk, D_IN), lambda i: (i, 0)),
            pl.BlockSpec((blk, D_IN), lambda i: (i, 0)),
            wspec, wspec, wspec, wspec,
        ],
        out_specs=[
            pl.BlockSpec((2, blk, H), lambda i: (0, i, 0)),
            pl.BlockSpec((2, blk, H), lambda i: (0, i, 0)),
        ],
        out_shape=[
            jax.ShapeDtypeStruct((2, NROWS, H), jnp.float32),
            jax.ShapeDtypeStruct((2, NROWS, H), jnp.float32),
        ],
    )(x_p, x_u, wvl, wrl, wvr, wrr)


def _pipeline_segsum(y_hbm, src_v, dst_v, rows0, rows1, acc_sh,
                     semg0, semg1, sems0, sems1, count=None):
    if count is not None:
        ones_v, cnt_sh, semc = count

    def fire_gather(j, rows, semg):
        pltpu.async_copy(y_hbm.at[src_v.at[j]], rows, semg)

    def wait_gather(j, rows, semg):
        pltpu.make_async_copy(y_hbm.at[src_v.at[j]], rows, semg).wait()

    def fire_scatter(j, rows, sems):
        pltpu.async_copy(rows, acc_sh.at[dst_v.at[j]], sems, add=True)

    def wait_scatter(j, rows, sems):
        pltpu.make_async_copy(rows, acc_sh.at[dst_v.at[j]], sems).wait()

    def fire_count(j):
        pltpu.async_copy(ones_v, cnt_sh.at[dst_v.at[j]], semc, add=True)

    def wait_count(j):
        pltpu.make_async_copy(ones_v, cnt_sh.at[dst_v.at[j]], semc).wait()

    def drain_scatter(j, rows, semg):
        wait_gather(j, rows, semg)
        pltpu.sync_copy(rows, acc_sh.at[dst_v.at[j]], add=True)
        if count is not None:
            fire_count(j)

            @pl.when(j >= 1)
            def _():
                wait_count(j - 1)

    fire_gather(0, rows0, semg0)

    def pair_body(p, carry):
        j0 = 2 * p
        fire_gather(j0 + 1, rows1, semg1)
        drain_scatter(j0, rows0, semg0)
        fire_gather(j0 + 2, rows0, semg0)
        drain_scatter(j0 + 1, rows1, semg1)
        return carry

    lax.fori_loop(0, (NCH - 1) // 2, pair_body, 0)
    drain_scatter(NCH - 1, rows0, semg0)
    if count is not None:
        wait_count(NCH - 1)


def _node_state(acc_v, cnt_v, root_v, bias, g, k):
    cvec = jnp.maximum(cnt_v[pl.ds(g * 16, 16)], 1.0)
    r = g * 16 + k
    return jnp.maximum(acc_v[r] / cvec[k] + root_v[r] + bias, 0.0), r


@functools.lru_cache(maxsize=None)
def _make_layer1():
    out_type = [
        jax.ShapeDtypeStruct((NC, NROWS, H), jnp.float32),
        jax.ShapeDtypeStruct((NC, NROWS, H), jnp.float32),
        jax.ShapeDtypeStruct((NC, NROWS), jnp.float32),
    ]
    scratch = [
        pltpu.VMEM((NCH, CHUNK), jnp.int32),
        pltpu.VMEM((NCH, CHUNK), jnp.int32),
        pltpu.VMEM((CHUNK, H), jnp.float32),
        pltpu.VMEM((CHUNK, H), jnp.float32),
        pltpu.VMEM_SHARED((NROWS, H), jnp.float32),
        pltpu.SemaphoreType.DMA,
        pltpu.SemaphoreType.DMA,
        pltpu.SemaphoreType.DMA,
        pltpu.SemaphoreType.DMA,
        pltpu.SemaphoreType.DMA,
        pltpu.VMEM((CHUNK,), jnp.float32),
        pltpu.VMEM_SHARED((NROWS,), jnp.float32),
        pltpu.VMEM((ZROWS, H), jnp.float32),
        pltpu.VMEM((ZROWS, H), jnp.float32),
        pltpu.VMEM((ZROWS,), jnp.float32),
        pltpu.VMEM((H,), jnp.float32),
        pltpu.VMEM((H, H), jnp.float32),
        pltpu.VMEM((H, H), jnp.float32),
        pltpu.VMEM((ZROWS, H), jnp.float32),
        pltpu.VMEM((ZROWS, H), jnp.float32),
    ]

    def body(y_hbm, src_hbm, dst_hbm, zrow_hbm, z1_hbm, root_hbm, b_hbm,
             wy_hbm, wr_hbm, y2_out, r2_out, cnt_out,
             src_v, dst_v, rows0, rows1, acc_sh, semg0, semg1, sems0,
             sems1, semc, ones_v, cnt_sh, acc_v, root_v, cnt_v, b_v,
             wy_v, wr_v, y2_v, r2_v):
        c = lax.axis_index("c")
        s = lax.axis_index("s")
        base = s * ZROWS

        pltpu.sync_copy(src_hbm.at[c].at[s], src_v)
        pltpu.sync_copy(dst_hbm.at[c].at[s], dst_v)
        pltpu.sync_copy(zrow_hbm, acc_sh.at[pl.ds(base, ZROWS)])
        pltpu.sync_copy(z1_hbm, cnt_sh.at[pl.ds(base, ZROWS)])
        pltpu.sync_copy(root_hbm.at[c].at[pl.ds(base, ZROWS)], root_v)
        pltpu.sync_copy(b_hbm.at[c], b_v)
        pltpu.sync_copy(wy_hbm.at[c], wy_v)
        pltpu.sync_copy(wr_hbm.at[c], wr_v)
        for i in range(CHUNK // 16):
            ones_v[pl.ds(i * 16, 16)] = jnp.ones((16,), jnp.float32)
        plsc.subcore_barrier()

        _pipeline_segsum(y_hbm, src_v, dst_v, rows0, rows1, acc_sh,
                         semg0, semg1, sems0, sems1,
                         count=(ones_v, cnt_sh, semc))
        plsc.subcore_barrier()

        pltpu.sync_copy(cnt_sh.at[pl.ds(base, ZROWS)],
                        cnt_out.at[c].at[pl.ds(base, ZROWS)])
        pltpu.sync_copy(cnt_sh.at[pl.ds(base, ZROWS)], cnt_v)
        pltpu.sync_copy(acc_sh.at[pl.ds(base, ZROWS)], acc_v)
        bias = b_v[...]

        def grp_body(g, carry):
            for k in range(16):
                u, r = _node_state(acc_v, cnt_v, root_v, bias, g, k)
                y2a = u[0] * wy_v[0]
                r2a = u[0] * wr_v[0]
                for k2 in range(1, 16):
                    y2a = y2a + u[k2] * wy_v[k2]
                    r2a = r2a + u[k2] * wr_v[k2]
                y2_v[r] = y2a
                r2_v[r] = r2a
            return carry

        lax.fori_loop(0, ZROWS // 16, grp_body, 0)
        pltpu.sync_copy(y2_v, y2_out.at[1 - c].at[pl.ds(base, ZROWS)])
        pltpu.sync_copy(r2_v, r2_out.at[c].at[pl.ds(base, ZROWS)])

    mesh = plsc.VectorSubcoreMesh(core_axis_name="c", subcore_axis_name="s",
                                  num_cores=NC, num_subcores=NS)
    return pl.kernel(body, out_type=out_type, mesh=mesh,
                     scratch_types=scratch,
                     compiler_params=pltpu.CompilerParams(
                         use_tc_tiling_on_sc=False))


@functools.lru_cache(maxsize=None)
def _make_layer2():
    out_type = [jax.ShapeDtypeStruct((NC, NROWS, H), jnp.float32)]
    scratch = [
        pltpu.VMEM((NCH, CHUNK), jnp.int32),
        pltpu.VMEM((NCH, CHUNK), jnp.int32),
        pltpu.VMEM((CHUNK, H), jnp.float32),
        pltpu.VMEM((CHUNK, H), jnp.float32),
        pltpu.VMEM_SHARED((NROWS, H), jnp.float32),
        pltpu.SemaphoreType.DMA,
        pltpu.SemaphoreType.DMA,
        pltpu.SemaphoreType.DMA,
        pltpu.SemaphoreType.DMA,
        pltpu.VMEM((ZROWS, H), jnp.float32),
        pltpu.VMEM((ZROWS, H), jnp.float32),
        pltpu.VMEM((ZROWS,), jnp.float32),
        pltpu.VMEM((H,), jnp.float32),
    ]

    def body(y_hbm, src_hbm, dst_hbm, zrow_hbm, cnt_hbm, root_hbm, b_hbm,
             s_out, src_v, dst_v, rows0, rows1, acc_sh, semg0, semg1,
             sems0, sems1, acc_v, root_v, cnt_v, b_v):
        c = lax.axis_index("c")
        s = lax.axis_index("s")
        base = s * ZROWS

        pltpu.sync_copy(src_hbm.at[c].at[s], src_v)
        pltpu.sync_copy(dst_hbm.at[c].at[s], dst_v)
        pltpu.sync_copy(zrow_hbm, acc_sh.at[pl.ds(base, ZROWS)])
        pltpu.sync_copy(cnt_hbm.at[c].at[pl.ds(base, ZROWS)], cnt_v)
        pltpu.sync_copy(root_hbm.at[c].at[pl.ds(base, ZROWS)], root_v)
        pltpu.sync_copy(b_hbm.at[c], b_v)
        plsc.subcore_barrier()

        _pipeline_segsum(y_hbm, src_v, dst_v, rows0, rows1, acc_sh,
                         semg0, semg1, sems0, sems1)
        plsc.subcore_barrier()

        pltpu.sync_copy(acc_sh.at[pl.ds(base, ZROWS)], acc_v)
        bias = b_v[...]

        def grp_body(g, carry):
            for k in range(16):
                u, r = _node_state(acc_v, cnt_v, root_v, bias, g, k)
                acc_v[r] = u
            return carry

        lax.fori_loop(0, ZROWS // 16, grp_body, 0)
        pltpu.sync_copy(acc_v, s_out.at[c].at[pl.ds(base, ZROWS)])

    mesh = plsc.VectorSubcoreMesh(core_axis_name="c", subcore_axis_name="s",
                                  num_cores=NC, num_subcores=NS)
    return pl.kernel(body, out_type=out_type, mesh=mesh,
                     scratch_types=scratch,
                     compiler_params=pltpu.CompilerParams(
                         use_tc_tiling_on_sc=False))


def _pad_edges(idx, fill):
    pad = jnp.full((EP - E_EDGES,), fill, jnp.int32)
    return jnp.concatenate([idx.astype(jnp.int32), pad])


def kernel(x_Persona, x_Ubicacion, edge_index_visits, edge_index_rev,
           W1v_l, b1v, W1v_r, W1r_l, b1r, W1r_r,
           W2v_l, b2v, W2v_r, W2r_l, b2r, W2r_r):
    ev = edge_index_visits.reshape(2 * E_EDGES)
    er = edge_index_rev.reshape(2 * E_EDGES)
    src_all = jnp.stack([
        _pad_edges(ev[:E_EDGES], 0),
        _pad_edges(er[:E_EDGES] + NROWS, NROWS),
    ]).reshape(NC, NS, NCH, CHUNK)
    dst_all = jnp.stack([
        _pad_edges(ev[E_EDGES:], TRASH),
        _pad_edges(er[E_EDGES:], TRASH),
    ]).reshape(NC, NS, NCH, CHUNK)
    zrow = jnp.zeros((ZROWS, H), jnp.float32)
    z1 = jnp.zeros((ZROWS,), jnp.float32)
    b1 = jnp.stack([b1v, b1r])
    b2 = jnp.stack([b2v, b2r])
    wy = jnp.stack([W2r_l, W2v_l])
    wr = jnp.stack([W2v_r, W2r_r])

    y1, r1 = _proj1(x_Persona, x_Ubicacion, W1v_l, W1r_l, W1v_r, W1r_r)

    y2, r2, cnt = _make_layer1()(y1.reshape(NC * NROWS, H), src_all,
                                 dst_all, zrow, z1, r1, b1, wy, wr)

    (out,) = _make_layer2()(y2.reshape(NC * NROWS, H), src_all, dst_all,
                            zrow, cnt, r2, b2)
    return (out[1, :N_NODES], out[0, :N_NODES])
